# Initial kernel scaffold; baseline (speedup 1.0000x reference)
import jax
import jax.numpy as jnp
from jax.experimental import pallas as pl


def kernel(x_pa, x_la, edge_index_pa, edge_index_la, edge_index_la_pa, edge_index_pa_la, edge_attr_pa, edge_attr_la, edge_attr_la_pa, edge_attr_pa_la, batch_pa, batch_la, params):
    raise NotImplementedError("write your pallas kernel here")


# speedup vs baseline: 1.0017x; 1.0017x over previous
import functools

import jax
import jax.numpy as jnp
from jax.experimental import pallas as pl
from jax.experimental.pallas import tpu as pltpu

H = 64
NG = 128


def _gru(h, m, p):
    gi = m @ p['W'] + p['bi']
    gh = h @ p['U'] + p['bh']
    i_r, i_z, i_n = jnp.split(gi, 3, axis=-1)
    h_r, h_z, h_n = jnp.split(gh, 3, axis=-1)
    r = jax.nn.sigmoid(i_r + h_r)
    z = jax.nn.sigmoid(i_z + h_z)
    n = jnp.tanh(i_n + r * h_n)
    return (1.0 - z) * n + z * h


def _segment_softmax(logits, seg, num):
    m = jax.ops.segment_max(logits, seg, num)
    ex = jnp.exp(logits - m[seg])
    den = jax.ops.segment_sum(ex, seg, num)
    return ex / (den[seg] + 1e-16)


def _gate_conv(x_src, x_dst, ei, ea, p, n_dst):
    src = ei[0]; dst = ei[1]
    h = x_dst @ p['lin']
    xj = (x_src @ p['src'])[src] + ea @ p['edge']
    gate = jax.nn.sigmoid(xj @ p['gate_w'] + p['gate_b'])
    msg = gate * jnp.tanh(xj @ p['msg_w'] + p['msg_b'])
    agg = jax.ops.segment_sum(msg, dst, n_dst)
    return _gru(h, agg, p['gru'])


def _gat_conv(x_src, x_dst, ei, p, n_dst):
    src = ei[0]; dst = ei[1]
    q = x_src @ p['Wsrc']
    k = x_dst @ p['Wdst']
    logit = jax.nn.leaky_relu((q[src] * p['a_src']).sum(-1) + (k[dst] * p['a_dst']).sum(-1), 0.2)
    alpha = _segment_softmax(logit, dst, n_dst)
    agg = jax.ops.segment_sum(alpha[:, None] * q[src], dst, n_dst)
    return _gru(x_dst, agg, p['gru'])


def _mol_conv(x_atoms, mol, batch, p, num_graphs):
    q = x_atoms @ p['Wsrc']
    k = mol @ p['Wdst']
    logit = jax.nn.leaky_relu((q * p['a_src']).sum(-1) + (k[batch] * p['a_dst']).sum(-1), 0.2)
    alpha = _segment_softmax(logit, batch, num_graphs)
    agg = jax.ops.segment_sum(alpha[:, None] * q, batch, num_graphs)
    return _gru(mol, agg, p['gru'])


def _final_mlp_body(mol_pa_ref, mol_la_ref, wpa_ref, bpa_ref, wla_ref, bla_ref,
                    w0_ref, b0_ref, w1_ref, b1_ref, out_ref):
    y_pa = mol_pa_ref[...] @ wpa_ref[...] + bpa_ref[...]
    y_la = mol_la_ref[...] @ wla_ref[...] + bla_ref[...]
    z = jnp.concatenate([y_pa, y_la], axis=-1)
    z = jax.nn.relu(z @ w0_ref[...] + b0_ref[...])
    out_ref[...] = z @ w1_ref[...] + b1_ref[...]


def _final_mlp(mol_pa, mol_la, params):
    return pl.pallas_call(
        _final_mlp_body,
        out_shape=jax.ShapeDtypeStruct((NG, 1), jnp.float32),
    )(mol_pa, mol_la,
      params['lin_pa']['w'], params['lin_pa']['b'].reshape(1, H),
      params['lin_la']['w'], params['lin_la']['b'].reshape(1, H),
      params['mlp0']['w'], params['mlp0']['b'].reshape(1, H),
      params['mlp1']['w'], params['mlp1']['b'].reshape(1, 1))


def kernel(x_pa, x_la, edge_index_pa, edge_index_la, edge_index_la_pa, edge_index_pa_la,
           edge_attr_pa, edge_attr_la, edge_attr_la_pa, edge_attr_pa_la,
           batch_pa, batch_la, params):
    ei_pa = jnp.asarray(edge_index_pa).astype(jnp.int32)
    ei_la = jnp.asarray(edge_index_la).astype(jnp.int32)
    ei_lp = jnp.asarray(edge_index_la_pa).astype(jnp.int32)
    ei_pl = jnp.asarray(edge_index_pa_la).astype(jnp.int32)
    batch_pa = jnp.asarray(batch_pa).astype(jnp.int32)
    batch_la = jnp.asarray(batch_la).astype(jnp.int32)
    n_pa = x_pa.shape[0]; n_la = x_la.shape[0]

    p1 = params['l1']
    h_pa = _gate_conv(x_pa, x_pa, ei_pa, edge_attr_pa, p1['pa'], n_pa) + \
        _gate_conv(x_la, x_pa, ei_lp, edge_attr_la_pa, p1['la_pa'], n_pa)
    h_la = _gate_conv(x_la, x_la, ei_la, edge_attr_la, p1['la'], n_la) + \
        _gate_conv(x_pa, x_la, ei_pl, edge_attr_pa_la, p1['pa_la'], n_la)
    h_pa = jax.nn.relu(h_pa); h_la = jax.nn.relu(h_la)
    for lname in ('l2', 'l3'):
        pll = params[lname]
        new_pa = _gat_conv(h_pa, h_pa, ei_pa, pll['pa'], n_pa) + \
            _gat_conv(h_la, h_pa, ei_lp, pll['la_pa'], n_pa)
        new_la = _gat_conv(h_la, h_la, ei_la, pll['la'], n_la) + \
            _gat_conv(h_pa, h_la, ei_pl, pll['pa_la'], n_la)
        h_pa = jax.nn.relu(new_pa); h_la = jax.nn.relu(new_la)
    mol_pa = jax.ops.segment_sum(h_pa, batch_pa, NG)
    mol_la = jax.ops.segment_sum(h_la, batch_la, NG)
    for _ in range(3):
        mol_pa = _mol_conv(h_pa, mol_pa, batch_pa, params['mol_pa'], NG)
        mol_la = _mol_conv(h_la, mol_la, batch_la, params['mol_la'], NG)
    return _final_mlp(mol_pa, mol_la, params)


# speedup vs baseline: 2.5139x; 2.5096x over previous
import functools

import jax
import jax.numpy as jnp
from jax import lax
from jax.experimental import pallas as pl
from jax.experimental.pallas import tpu as pltpu
from jax.experimental.pallas import tpu_sc as plsc

H = 64
NG = 128
NW = 32
CH = 128


@functools.lru_cache(maxsize=None)
def _make_sc_gather(E, D, N):
    per_tile = E // NW
    nchunks = per_tile // CH
    mesh = plsc.VectorSubcoreMesh(core_axis_name="c", subcore_axis_name="s")

    @functools.partial(
        pl.kernel, mesh=mesh,
        out_type=jax.ShapeDtypeStruct((E, D), jnp.float32),
        compiler_params=pltpu.CompilerParams(use_tc_tiling_on_sc=False),
        scratch_types=[
            pltpu.VMEM((CH,), jnp.int32),
            pltpu.VMEM((CH, D), jnp.float32),
            pltpu.SemaphoreType.DMA,
        ],
    )
    def gather_k(table_hbm, idx_hbm, out_hbm, idx_v, rows_v, sem):
        wid = lax.axis_index("s") * 2 + lax.axis_index("c")
        base0 = wid * per_tile

        def body(c, carry):
            base = base0 + c * CH
            pltpu.sync_copy(idx_hbm.at[pl.ds(base, CH)], idx_v)
            pltpu.async_copy(table_hbm.at[idx_v], rows_v, sem).wait()
            pltpu.sync_copy(rows_v, out_hbm.at[pl.ds(base, CH)])
            return carry

        lax.fori_loop(0, nchunks, body, 0)

    return gather_k


def _sc_gather(table, idx_pad):
    E = idx_pad.shape[0]
    N, D = table.shape
    return _make_sc_gather(E, D, N)(table, idx_pad)


def _acc_rows(n):
    return -(-(n + 1) // (16 * CH)) * (16 * CH)


@functools.lru_cache(maxsize=None)
def _make_sc_scatter_add(E, N, D):
    NP = _acc_rows(N)
    per_tile = E // 16
    nchunks = per_tile // CH
    zcopies = NP // 16 // CH
    mesh = plsc.VectorSubcoreMesh(core_axis_name="c", subcore_axis_name="s")

    @functools.partial(
        pl.kernel, mesh=mesh,
        out_type=jax.ShapeDtypeStruct((2 * NP, D), jnp.float32),
        compiler_params=pltpu.CompilerParams(use_tc_tiling_on_sc=False),
        scratch_types=(
            pltpu.VMEM((CH,), jnp.int32),
            pltpu.VMEM((CH, D), jnp.float32),
            pltpu.VMEM((CH, D), jnp.float32),
            pltpu.VMEM_SHARED((NP, D), jnp.float32),
            pltpu.SemaphoreType.DMA,
        ),
    )
    def scatter_k(vals_hbm, idx_hbm, out_hbm, idx_v, val_v, zero_v, acc_sh, sem):
        core = lax.axis_index("c")
        sub = lax.axis_index("s")

        def zrow(r, carry):
            for j in range(D // 16):
                zero_v[r, pl.ds(j * 16, 16)] = jnp.zeros((16,), jnp.float32)
            return carry
        lax.fori_loop(0, CH, zrow, 0)

        zbase = sub * (NP // 16)
        def zacc(i, carry):
            pltpu.sync_copy(zero_v, acc_sh.at[pl.ds(zbase + i * CH, CH)])
            return carry
        lax.fori_loop(0, zcopies, zacc, 0)
        plsc.subcore_barrier()

        def body(c, carry):
            base = sub * per_tile + c * CH
            pltpu.sync_copy(idx_hbm.at[pl.ds(base, CH)], idx_v)
            pltpu.sync_copy(vals_hbm.at[pl.ds(core * E + base, CH)], val_v)
            pltpu.sync_copy(val_v, acc_sh.at[idx_v], add=True)
            return carry
        lax.fori_loop(0, nchunks, body, 0)
        plsc.subcore_barrier()

        def cpout(i, carry):
            sl = pl.ds(zbase + i * CH, CH)
            pltpu.sync_copy(acc_sh.at[sl], val_v)
            pltpu.sync_copy(val_v, out_hbm.at[pl.ds(core * NP + zbase + i * CH, CH)])
            return carry
        lax.fori_loop(0, zcopies, cpout, 0)

    return scatter_k


def _sc_segment_sum(vals64, dst_pad, n, w=None):
    Ep = vals64.shape[0]
    NP = _acc_rows(n)
    vals = jnp.concatenate([vals64[:, :32], vals64[:, 32:]], axis=0)
    acc = _make_sc_scatter_add(Ep, n, 32)(vals, dst_pad)
    agg = jnp.concatenate([acc[:n], acc[NP:NP + n]], axis=1)
    den = None
    if w is not None:
        w16 = jnp.concatenate([w[:, None], jnp.zeros((Ep, 15), jnp.float32)], axis=1)
        dacc = _make_sc_scatter_add(Ep, n, 16)(jnp.concatenate([w16, w16], axis=0), dst_pad)
        den = dacc[:n, 0]
    return agg, den


def _pad_edges(ei, n_dst):
    E = ei.shape[1]
    Ep = -(-E // (NW * CH)) * (NW * CH)
    src = jnp.concatenate([ei[0], jnp.zeros((Ep - E,), jnp.int32)])
    dst = jnp.concatenate([ei[1], jnp.full((Ep - E,), n_dst, jnp.int32)])
    return src, dst, E, Ep


def _gru(h, m, p):
    gi = m @ p['W'] + p['bi']
    gh = h @ p['U'] + p['bh']
    i_r, i_z, i_n = jnp.split(gi, 3, axis=-1)
    h_r, h_z, h_n = jnp.split(gh, 3, axis=-1)
    r = jax.nn.sigmoid(i_r + h_r)
    z = jax.nn.sigmoid(i_z + h_z)
    n = jnp.tanh(i_n + r * h_n)
    return (1.0 - z) * n + z * h


def _segment_softmax(logits, seg, num):
    m = jax.ops.segment_max(logits, seg, num)
    ex = jnp.exp(logits - m[seg])
    den = jax.ops.segment_sum(ex, seg, num)
    return ex / (den[seg] + 1e-16)


def _gate_conv(x_src, x_dst, ei, ea, p, n_dst):
    src, dst, E, Ep = _pad_edges(ei, n_dst)
    h = x_dst @ p['lin']
    xj = _sc_gather(x_src @ p['src'], src)[:E] + ea @ p['edge']
    gate = jax.nn.sigmoid(xj @ p['gate_w'] + p['gate_b'])
    msg = gate * jnp.tanh(xj @ p['msg_w'] + p['msg_b'])
    msg = jnp.concatenate([msg, jnp.zeros((Ep - E, H), jnp.float32)], axis=0)
    agg, _ = _sc_segment_sum(msg, dst, n_dst)
    return _gru(h, agg, p['gru'])


def _gat_conv(x_src, x_dst, ei, p, n_dst):
    src, dst, E, Ep = _pad_edges(ei, n_dst)
    q = x_src @ p['Wsrc']
    qa = q @ p['a_src']
    ka = (x_dst @ p['Wdst']) @ p['a_dst']
    qext = jnp.concatenate(
        [q, qa[:, None], jnp.zeros((q.shape[0], 15), jnp.float32)], axis=1)
    kext = jnp.concatenate(
        [ka[:, None], jnp.zeros((ka.shape[0], 15), jnp.float32)], axis=1)
    kext = jnp.concatenate([kext, jnp.zeros((16, 16), jnp.float32)], axis=0)
    qg = _sc_gather(qext, src)
    kg = _sc_gather(kext, dst)
    logit = qg[:, 64] + kg[:, 0]
    w = jnp.exp(jnp.where(logit >= 0, logit, 0.2 * logit))
    msg = w[:, None] * qg[:, :64]
    num, den = _sc_segment_sum(msg, dst, n_dst, w=w)
    agg = num / (den[:, None] + 1e-16)
    return _gru(x_dst, agg, p['gru'])


def _mol_conv(x_atoms, mol, batch, p, num_graphs):
    q = x_atoms @ p['Wsrc']
    k = mol @ p['Wdst']
    logit = jax.nn.leaky_relu((q * p['a_src']).sum(-1) + (k[batch] * p['a_dst']).sum(-1), 0.2)
    alpha = _segment_softmax(logit, batch, num_graphs)
    agg = jax.ops.segment_sum(alpha[:, None] * q, batch, num_graphs)
    return _gru(mol, agg, p['gru'])


def _final_mlp_body(mol_pa_ref, mol_la_ref, wpa_ref, bpa_ref, wla_ref, bla_ref,
                    w0_ref, b0_ref, w1_ref, b1_ref, out_ref):
    y_pa = mol_pa_ref[...] @ wpa_ref[...] + bpa_ref[...]
    y_la = mol_la_ref[...] @ wla_ref[...] + bla_ref[...]
    z = jnp.concatenate([y_pa, y_la], axis=-1)
    z = jax.nn.relu(z @ w0_ref[...] + b0_ref[...])
    out_ref[...] = z @ w1_ref[...] + b1_ref[...]


def _final_mlp(mol_pa, mol_la, params):
    return pl.pallas_call(
        _final_mlp_body,
        out_shape=jax.ShapeDtypeStruct((NG, 1), jnp.float32),
    )(mol_pa, mol_la,
      params['lin_pa']['w'], params['lin_pa']['b'].reshape(1, H),
      params['lin_la']['w'], params['lin_la']['b'].reshape(1, H),
      params['mlp0']['w'], params['mlp0']['b'].reshape(1, H),
      params['mlp1']['w'], params['mlp1']['b'].reshape(1, 1))


def kernel(x_pa, x_la, edge_index_pa, edge_index_la, edge_index_la_pa, edge_index_pa_la,
           edge_attr_pa, edge_attr_la, edge_attr_la_pa, edge_attr_pa_la,
           batch_pa, batch_la, params):
    ei_pa = jnp.asarray(edge_index_pa).astype(jnp.int32)
    ei_la = jnp.asarray(edge_index_la).astype(jnp.int32)
    ei_lp = jnp.asarray(edge_index_la_pa).astype(jnp.int32)
    ei_pl = jnp.asarray(edge_index_pa_la).astype(jnp.int32)
    batch_pa = jnp.asarray(batch_pa).astype(jnp.int32)
    batch_la = jnp.asarray(batch_la).astype(jnp.int32)
    n_pa = x_pa.shape[0]; n_la = x_la.shape[0]

    p1 = params['l1']
    h_pa = _gate_conv(x_pa, x_pa, ei_pa, edge_attr_pa, p1['pa'], n_pa) + \
        _gate_conv(x_la, x_pa, ei_lp, edge_attr_la_pa, p1['la_pa'], n_pa)
    h_la = _gate_conv(x_la, x_la, ei_la, edge_attr_la, p1['la'], n_la) + \
        _gate_conv(x_pa, x_la, ei_pl, edge_attr_pa_la, p1['pa_la'], n_la)
    h_pa = jax.nn.relu(h_pa); h_la = jax.nn.relu(h_la)
    for lname in ('l2', 'l3'):
        pll = params[lname]
        new_pa = _gat_conv(h_pa, h_pa, ei_pa, pll['pa'], n_pa) + \
            _gat_conv(h_la, h_pa, ei_lp, pll['la_pa'], n_pa)
        new_la = _gat_conv(h_la, h_la, ei_la, pll['la'], n_la) + \
            _gat_conv(h_pa, h_la, ei_pl, pll['pa_la'], n_la)
        h_pa = jax.nn.relu(new_pa); h_la = jax.nn.relu(new_la)
    mol_pa = jax.ops.segment_sum(h_pa, batch_pa, NG)
    mol_la = jax.ops.segment_sum(h_la, batch_la, NG)
    for _ in range(3):
        mol_pa = _mol_conv(h_pa, mol_pa, batch_pa, params['mol_pa'], NG)
        mol_la = _mol_conv(h_la, mol_la, batch_la, params['mol_la'], NG)
    return _final_mlp(mol_pa, mol_la, params)


# speedup vs baseline: 2.8235x; 1.1232x over previous
import functools

import jax
import jax.numpy as jnp
from jax import lax
from jax.experimental import pallas as pl
from jax.experimental.pallas import tpu as pltpu
from jax.experimental.pallas import tpu_sc as plsc

H = 64
NG = 128
NW = 32
CH = 128


@functools.lru_cache(maxsize=None)
def _make_sc_gather(E, D, N):
    per_tile = E // NW
    nchunks = per_tile // CH
    mesh = plsc.VectorSubcoreMesh(core_axis_name="c", subcore_axis_name="s")

    @functools.partial(
        pl.kernel, mesh=mesh,
        out_type=jax.ShapeDtypeStruct((E, D), jnp.float32),
        compiler_params=pltpu.CompilerParams(use_tc_tiling_on_sc=False),
        scratch_types=[
            pltpu.VMEM((CH,), jnp.int32),
            pltpu.VMEM((CH, D), jnp.float32),
            pltpu.SemaphoreType.DMA,
        ],
    )
    def gather_k(table_hbm, idx_hbm, out_hbm, idx_v, rows_v, sem):
        wid = lax.axis_index("s") * 2 + lax.axis_index("c")
        base0 = wid * per_tile

        def body(c, carry):
            base = base0 + c * CH
            pltpu.sync_copy(idx_hbm.at[pl.ds(base, CH)], idx_v)
            pltpu.async_copy(table_hbm.at[idx_v], rows_v, sem).wait()
            pltpu.sync_copy(rows_v, out_hbm.at[pl.ds(base, CH)])
            return carry

        lax.fori_loop(0, nchunks, body, 0)

    return gather_k


def _sc_gather(table, idx_pad):
    E = idx_pad.shape[0]
    N, D = table.shape
    return _make_sc_gather(E, D, N)(table, idx_pad)


def _acc_rows(n):
    return -(-(n + 1) // (16 * CH)) * (16 * CH)


@functools.lru_cache(maxsize=None)
def _make_sc_scatter_add(E, N, D):
    NP = _acc_rows(N)
    per_tile = E // 16
    nchunks = per_tile // CH
    zcopies = NP // 16 // CH
    mesh = plsc.VectorSubcoreMesh(core_axis_name="c", subcore_axis_name="s")

    @functools.partial(
        pl.kernel, mesh=mesh,
        out_type=jax.ShapeDtypeStruct((2 * NP, D), jnp.float32),
        compiler_params=pltpu.CompilerParams(use_tc_tiling_on_sc=False),
        scratch_types=(
            pltpu.VMEM((CH,), jnp.int32),
            pltpu.VMEM((CH, D), jnp.float32),
            pltpu.VMEM((CH, D), jnp.float32),
            pltpu.VMEM_SHARED((NP, D), jnp.float32),
            pltpu.SemaphoreType.DMA,
        ),
    )
    def scatter_k(vals_hbm, idx_hbm, out_hbm, idx_v, val_v, zero_v, acc_sh, sem):
        core = lax.axis_index("c")
        sub = lax.axis_index("s")

        def zrow(r, carry):
            for j in range(D // 16):
                zero_v[r, pl.ds(j * 16, 16)] = jnp.zeros((16,), jnp.float32)
            return carry
        lax.fori_loop(0, CH, zrow, 0)

        zbase = sub * (NP // 16)
        def zacc(i, carry):
            pltpu.sync_copy(zero_v, acc_sh.at[pl.ds(zbase + i * CH, CH)])
            return carry
        lax.fori_loop(0, zcopies, zacc, 0)
        plsc.subcore_barrier()

        def body(c, carry):
            base = sub * per_tile + c * CH
            pltpu.sync_copy(idx_hbm.at[pl.ds(base, CH)], idx_v)
            pltpu.sync_copy(vals_hbm.at[pl.ds(core * E + base, CH)], val_v)
            pltpu.sync_copy(val_v, acc_sh.at[idx_v], add=True)
            return carry
        lax.fori_loop(0, nchunks, body, 0)
        plsc.subcore_barrier()

        def cpout(i, carry):
            sl = pl.ds(zbase + i * CH, CH)
            pltpu.sync_copy(acc_sh.at[sl], val_v)
            pltpu.sync_copy(val_v, out_hbm.at[pl.ds(core * NP + zbase + i * CH, CH)])
            return carry
        lax.fori_loop(0, zcopies, cpout, 0)

    return scatter_k


def _sc_segment_sum(vals64, dst_pad, n, w=None):
    Ep = vals64.shape[0]
    NP = _acc_rows(n)
    vals = jnp.concatenate([vals64[:, :32], vals64[:, 32:]], axis=0)
    acc = _make_sc_scatter_add(Ep, n, 32)(vals, dst_pad)
    agg = jnp.concatenate([acc[:n], acc[NP:NP + n]], axis=1)
    den = None
    if w is not None:
        w16 = jnp.concatenate([w[:, None], jnp.zeros((Ep, 15), jnp.float32)], axis=1)
        dacc = _make_sc_scatter_add(Ep, n, 16)(jnp.concatenate([w16, w16], axis=0), dst_pad)
        den = dacc[:n, 0]
    return agg, den


def _pad_edges(ei, n_dst):
    E = ei.shape[1]
    Ep = -(-E // (NW * CH)) * (NW * CH)
    src = jnp.concatenate([ei[0], jnp.zeros((Ep - E,), jnp.int32)])
    dst = jnp.concatenate([ei[1], jnp.full((Ep - E,), n_dst, jnp.int32)])
    return src, dst, E, Ep


def _gru(h, m, p):
    gi = m @ p['W'] + p['bi']
    gh = h @ p['U'] + p['bh']
    i_r, i_z, i_n = jnp.split(gi, 3, axis=-1)
    h_r, h_z, h_n = jnp.split(gh, 3, axis=-1)
    r = jax.nn.sigmoid(i_r + h_r)
    z = jax.nn.sigmoid(i_z + h_z)
    n = jnp.tanh(i_n + r * h_n)
    return (1.0 - z) * n + z * h


def _segment_softmax(logits, seg, num):
    m = jax.ops.segment_max(logits, seg, num)
    ex = jnp.exp(logits - m[seg])
    den = jax.ops.segment_sum(ex, seg, num)
    return ex / (den[seg] + 1e-16)


def _gate_conv(x_src, x_dst, ei, ea, p, n_dst):
    src, dst, E, Ep = _pad_edges(ei, n_dst)
    h = x_dst @ p['lin']
    xj = _sc_gather(x_src @ p['src'], src)[:E] + ea @ p['edge']
    gate = jax.nn.sigmoid(xj @ p['gate_w'] + p['gate_b'])
    msg = gate * jnp.tanh(xj @ p['msg_w'] + p['msg_b'])
    msg = jnp.concatenate([msg, jnp.zeros((Ep - E, H), jnp.float32)], axis=0)
    agg, _ = _sc_segment_sum(msg, dst, n_dst)
    return _gru(h, agg, p['gru'])


def _gat_conv(x_src, x_dst, ei, p, n_dst):
    src, dst, E, Ep = _pad_edges(ei, n_dst)
    q = x_src @ p['Wsrc']
    qa = q @ p['a_src']
    ka = (x_dst @ p['Wdst']) @ p['a_dst']
    qext = jnp.concatenate(
        [q, qa[:, None], jnp.zeros((q.shape[0], 15), jnp.float32)], axis=1)
    kext = jnp.concatenate(
        [ka[:, None], jnp.zeros((ka.shape[0], 15), jnp.float32)], axis=1)
    kext = jnp.concatenate([kext, jnp.zeros((16, 16), jnp.float32)], axis=0)
    qg = _sc_gather(qext, src)
    kg = _sc_gather(kext, dst)
    logit = qg[:, 64] + kg[:, 0]
    w = jnp.exp(jnp.where(logit >= 0, logit, 0.2 * logit))
    msg = w[:, None] * qg[:, :64]
    num, den = _sc_segment_sum(msg, dst, n_dst, w=w)
    agg = num / (den[:, None] + 1e-16)
    return _gru(x_dst, agg, p['gru'])


def _mol_conv(q, qa, onehot, mol, p, num_graphs):
    kb = (mol @ p['Wdst']) @ p['a_dst']
    logit = qa + onehot @ kb
    w = jnp.exp(jnp.where(logit >= 0, logit, 0.2 * logit))
    nd = onehot.T @ jnp.concatenate([w[:, None] * q, w[:, None]], axis=1)
    agg = nd[:, :H] / (nd[:, H:] + 1e-16)
    return _gru(mol, agg, p['gru'])


def _final_mlp_body(mol_pa_ref, mol_la_ref, wpa_ref, bpa_ref, wla_ref, bla_ref,
                    w0_ref, b0_ref, w1_ref, b1_ref, out_ref):
    y_pa = mol_pa_ref[...] @ wpa_ref[...] + bpa_ref[...]
    y_la = mol_la_ref[...] @ wla_ref[...] + bla_ref[...]
    z = jnp.concatenate([y_pa, y_la], axis=-1)
    z = jax.nn.relu(z @ w0_ref[...] + b0_ref[...])
    out_ref[...] = z @ w1_ref[...] + b1_ref[...]


def _final_mlp(mol_pa, mol_la, params):
    return pl.pallas_call(
        _final_mlp_body,
        out_shape=jax.ShapeDtypeStruct((NG, 1), jnp.float32),
    )(mol_pa, mol_la,
      params['lin_pa']['w'], params['lin_pa']['b'].reshape(1, H),
      params['lin_la']['w'], params['lin_la']['b'].reshape(1, H),
      params['mlp0']['w'], params['mlp0']['b'].reshape(1, H),
      params['mlp1']['w'], params['mlp1']['b'].reshape(1, 1))


def kernel(x_pa, x_la, edge_index_pa, edge_index_la, edge_index_la_pa, edge_index_pa_la,
           edge_attr_pa, edge_attr_la, edge_attr_la_pa, edge_attr_pa_la,
           batch_pa, batch_la, params):
    ei_pa = jnp.asarray(edge_index_pa).astype(jnp.int32)
    ei_la = jnp.asarray(edge_index_la).astype(jnp.int32)
    ei_lp = jnp.asarray(edge_index_la_pa).astype(jnp.int32)
    ei_pl = jnp.asarray(edge_index_pa_la).astype(jnp.int32)
    batch_pa = jnp.asarray(batch_pa).astype(jnp.int32)
    batch_la = jnp.asarray(batch_la).astype(jnp.int32)
    n_pa = x_pa.shape[0]; n_la = x_la.shape[0]

    p1 = params['l1']
    h_pa = _gate_conv(x_pa, x_pa, ei_pa, edge_attr_pa, p1['pa'], n_pa) + \
        _gate_conv(x_la, x_pa, ei_lp, edge_attr_la_pa, p1['la_pa'], n_pa)
    h_la = _gate_conv(x_la, x_la, ei_la, edge_attr_la, p1['la'], n_la) + \
        _gate_conv(x_pa, x_la, ei_pl, edge_attr_pa_la, p1['pa_la'], n_la)
    h_pa = jax.nn.relu(h_pa); h_la = jax.nn.relu(h_la)
    for lname in ('l2', 'l3'):
        pll = params[lname]
        new_pa = _gat_conv(h_pa, h_pa, ei_pa, pll['pa'], n_pa) + \
            _gat_conv(h_la, h_pa, ei_lp, pll['la_pa'], n_pa)
        new_la = _gat_conv(h_la, h_la, ei_la, pll['la'], n_la) + \
            _gat_conv(h_pa, h_la, ei_pl, pll['pa_la'], n_la)
        h_pa = jax.nn.relu(new_pa); h_la = jax.nn.relu(new_la)
    oh_pa = (batch_pa[:, None] == jnp.arange(NG, dtype=jnp.int32)).astype(jnp.float32)
    oh_la = (batch_la[:, None] == jnp.arange(NG, dtype=jnp.int32)).astype(jnp.float32)
    mol_pa = oh_pa.T @ h_pa
    mol_la = oh_la.T @ h_la
    q_pa = h_pa @ params['mol_pa']['Wsrc']; qa_pa = q_pa @ params['mol_pa']['a_src']
    q_la = h_la @ params['mol_la']['Wsrc']; qa_la = q_la @ params['mol_la']['a_src']
    for _ in range(3):
        mol_pa = _mol_conv(q_pa, qa_pa, oh_pa, mol_pa, params['mol_pa'], NG)
        mol_la = _mol_conv(q_la, qa_la, oh_la, mol_la, params['mol_la'], NG)
    return _final_mlp(mol_pa, mol_la, params)


# speedup vs baseline: 2.8358x; 1.0044x over previous
import functools

import jax
import jax.numpy as jnp
from jax import lax
from jax.experimental import pallas as pl
from jax.experimental.pallas import tpu as pltpu
from jax.experimental.pallas import tpu_sc as plsc

H = 64
NG = 128
NW = 32
CH = 128


@functools.lru_cache(maxsize=None)
def _make_sc_gather(E, D, N):
    SUB = 4
    BB = SUB * CH
    per_tile = E // NW
    nblocks = per_tile // BB
    mesh = plsc.VectorSubcoreMesh(core_axis_name="c", subcore_axis_name="s")

    @functools.partial(
        pl.kernel, mesh=mesh,
        out_type=jax.ShapeDtypeStruct((E, D), jnp.float32),
        compiler_params=pltpu.CompilerParams(use_tc_tiling_on_sc=False),
        scratch_types=[
            pltpu.VMEM((SUB, CH), jnp.int32),
            pltpu.VMEM((BB, D), jnp.float32),
            pltpu.SemaphoreType.DMA,
        ],
    )
    def gather_k(table_hbm, idx_hbm, out_hbm, idx_v, rows_v, sem):
        wid = lax.axis_index("s") * 2 + lax.axis_index("c")
        base0 = wid * per_tile

        def body(b, carry):
            base = base0 + b * BB
            pltpu.sync_copy(idx_hbm.at[pl.ds(base // CH, SUB)], idx_v)
            cps = [pltpu.async_copy(table_hbm.at[idx_v.at[j]],
                                    rows_v.at[pl.ds(j * CH, CH)], sem)
                   for j in range(SUB)]
            for cp in cps:
                cp.wait()
            pltpu.sync_copy(rows_v, out_hbm.at[pl.ds(base, BB)])
            return carry

        lax.fori_loop(0, nblocks, body, 0)

    return gather_k


def _sc_gather(table, idx_pad):
    E = idx_pad.shape[0]
    N, D = table.shape
    return _make_sc_gather(E, D, N)(table, idx_pad.reshape(E // CH, CH))


def _acc_rows(n):
    return -(-(n + 1) // (16 * CH)) * (16 * CH)


@functools.lru_cache(maxsize=None)
def _make_sc_scatter_add(E, N, D):
    NP = _acc_rows(N)
    SUB = 4
    BB = SUB * CH
    per_tile = E // 16
    nblocks = per_tile // BB
    zcopies = NP // 16 // CH
    mesh = plsc.VectorSubcoreMesh(core_axis_name="c", subcore_axis_name="s")

    @functools.partial(
        pl.kernel, mesh=mesh,
        out_type=jax.ShapeDtypeStruct((2 * NP, D), jnp.float32),
        compiler_params=pltpu.CompilerParams(use_tc_tiling_on_sc=False),
        scratch_types=(
            pltpu.VMEM((SUB, CH), jnp.int32),
            pltpu.VMEM((BB, D), jnp.float32),
            pltpu.VMEM((CH, D), jnp.float32),
            pltpu.VMEM_SHARED((NP, D), jnp.float32),
            pltpu.SemaphoreType.DMA,
        ),
    )
    def scatter_k(vals_hbm, idx_hbm, out_hbm, idx_v, val_v, zero_v, acc_sh, sem):
        core = lax.axis_index("c")
        sub = lax.axis_index("s")

        def zrow(r, carry):
            for j in range(D // 16):
                zero_v[r, pl.ds(j * 16, 16)] = jnp.zeros((16,), jnp.float32)
            return carry
        lax.fori_loop(0, CH, zrow, 0)

        zbase = sub * (NP // 16)
        def zacc(i, carry):
            pltpu.sync_copy(zero_v, acc_sh.at[pl.ds(zbase + i * CH, CH)])
            return carry
        lax.fori_loop(0, zcopies, zacc, 0)
        plsc.subcore_barrier()

        def body(b, carry):
            base = sub * per_tile + b * BB
            pltpu.sync_copy(idx_hbm.at[pl.ds(base // CH, SUB)], idx_v)
            pltpu.sync_copy(vals_hbm.at[pl.ds(core * E + base, BB)], val_v)
            cps = [pltpu.async_copy(val_v.at[pl.ds(j * CH, CH)],
                                    acc_sh.at[idx_v.at[j]], sem, add=True)
                   for j in range(SUB)]
            for cp in cps:
                cp.wait()
            return carry
        lax.fori_loop(0, nblocks, body, 0)
        plsc.subcore_barrier()

        def cpout(i, carry):
            sl = pl.ds(zbase + i * BB, BB)
            pltpu.sync_copy(acc_sh.at[sl], val_v)
            pltpu.sync_copy(val_v, out_hbm.at[pl.ds(core * NP + zbase + i * BB, BB)])
            return carry
        lax.fori_loop(0, zcopies // SUB, cpout, 0)
        for r in range(zcopies % SUB):
            off = (zcopies // SUB) * BB + r * CH
            pltpu.sync_copy(acc_sh.at[pl.ds(zbase + off, CH)], val_v.at[pl.ds(0, CH)])
            pltpu.sync_copy(val_v.at[pl.ds(0, CH)],
                            out_hbm.at[pl.ds(core * NP + zbase + off, CH)])

    return scatter_k


def _sc_segment_sum(vals64, dst_pad, n, w=None):
    Ep = vals64.shape[0]
    NP = _acc_rows(n)
    vals = jnp.concatenate([vals64[:, :32], vals64[:, 32:]], axis=0)
    acc = _make_sc_scatter_add(Ep, n, 32)(vals, dst_pad.reshape(Ep // CH, CH))
    agg = jnp.concatenate([acc[:n], acc[NP:NP + n]], axis=1)
    den = None
    if w is not None:
        w16 = jnp.concatenate([w[:, None], jnp.zeros((Ep, 15), jnp.float32)], axis=1)
        dacc = _make_sc_scatter_add(Ep, n, 16)(jnp.concatenate([w16, w16], axis=0), dst_pad.reshape(Ep // CH, CH))
        den = dacc[:n, 0]
    return agg, den


def _pad_edges(ei, n_dst):
    E = ei.shape[1]
    Ep = -(-E // 16384) * 16384
    src = jnp.concatenate([ei[0], jnp.zeros((Ep - E,), jnp.int32)])
    dst = jnp.concatenate([ei[1], jnp.full((Ep - E,), n_dst, jnp.int32)])
    return src, dst, E, Ep


def _gru(h, m, p):
    gi = m @ p['W'] + p['bi']
    gh = h @ p['U'] + p['bh']
    i_r, i_z, i_n = jnp.split(gi, 3, axis=-1)
    h_r, h_z, h_n = jnp.split(gh, 3, axis=-1)
    r = jax.nn.sigmoid(i_r + h_r)
    z = jax.nn.sigmoid(i_z + h_z)
    n = jnp.tanh(i_n + r * h_n)
    return (1.0 - z) * n + z * h


def _segment_softmax(logits, seg, num):
    m = jax.ops.segment_max(logits, seg, num)
    ex = jnp.exp(logits - m[seg])
    den = jax.ops.segment_sum(ex, seg, num)
    return ex / (den[seg] + 1e-16)


def _gate_conv(x_src, x_dst, ei, ea, p, n_dst):
    src, dst, E, Ep = _pad_edges(ei, n_dst)
    h = x_dst @ p['lin']
    xj = _sc_gather(x_src @ p['src'], src)[:E] + ea @ p['edge']
    gate = jax.nn.sigmoid(xj @ p['gate_w'] + p['gate_b'])
    msg = gate * jnp.tanh(xj @ p['msg_w'] + p['msg_b'])
    msg = jnp.concatenate([msg, jnp.zeros((Ep - E, H), jnp.float32)], axis=0)
    agg, _ = _sc_segment_sum(msg, dst, n_dst)
    return _gru(h, agg, p['gru'])


def _gat_conv(x_src, x_dst, ei, p, n_dst):
    src, dst, E, Ep = _pad_edges(ei, n_dst)
    q = x_src @ p['Wsrc']
    qa = q @ p['a_src']
    ka = (x_dst @ p['Wdst']) @ p['a_dst']
    qext = jnp.concatenate(
        [q, qa[:, None], jnp.zeros((q.shape[0], 15), jnp.float32)], axis=1)
    kext = jnp.concatenate(
        [ka[:, None], jnp.zeros((ka.shape[0], 15), jnp.float32)], axis=1)
    kext = jnp.concatenate([kext, jnp.zeros((16, 16), jnp.float32)], axis=0)
    qg = _sc_gather(qext, src)
    kg = _sc_gather(kext, dst)
    logit = qg[:, 64] + kg[:, 0]
    w = jnp.exp(jnp.where(logit >= 0, logit, 0.2 * logit))
    msg = w[:, None] * qg[:, :64]
    num, den = _sc_segment_sum(msg, dst, n_dst, w=w)
    agg = num / (den[:, None] + 1e-16)
    return _gru(x_dst, agg, p['gru'])


def _mol_conv(q, qa, onehot, mol, p, num_graphs):
    kb = (mol @ p['Wdst']) @ p['a_dst']
    logit = qa + onehot @ kb
    w = jnp.exp(jnp.where(logit >= 0, logit, 0.2 * logit))
    nd = onehot.T @ jnp.concatenate([w[:, None] * q, w[:, None]], axis=1)
    agg = nd[:, :H] / (nd[:, H:] + 1e-16)
    return _gru(mol, agg, p['gru'])


def _final_mlp_body(mol_pa_ref, mol_la_ref, wpa_ref, bpa_ref, wla_ref, bla_ref,
                    w0_ref, b0_ref, w1_ref, b1_ref, out_ref):
    y_pa = mol_pa_ref[...] @ wpa_ref[...] + bpa_ref[...]
    y_la = mol_la_ref[...] @ wla_ref[...] + bla_ref[...]
    z = jnp.concatenate([y_pa, y_la], axis=-1)
    z = jax.nn.relu(z @ w0_ref[...] + b0_ref[...])
    out_ref[...] = z @ w1_ref[...] + b1_ref[...]


def _final_mlp(mol_pa, mol_la, params):
    return pl.pallas_call(
        _final_mlp_body,
        out_shape=jax.ShapeDtypeStruct((NG, 1), jnp.float32),
    )(mol_pa, mol_la,
      params['lin_pa']['w'], params['lin_pa']['b'].reshape(1, H),
      params['lin_la']['w'], params['lin_la']['b'].reshape(1, H),
      params['mlp0']['w'], params['mlp0']['b'].reshape(1, H),
      params['mlp1']['w'], params['mlp1']['b'].reshape(1, 1))


def kernel(x_pa, x_la, edge_index_pa, edge_index_la, edge_index_la_pa, edge_index_pa_la,
           edge_attr_pa, edge_attr_la, edge_attr_la_pa, edge_attr_pa_la,
           batch_pa, batch_la, params):
    ei_pa = jnp.asarray(edge_index_pa).astype(jnp.int32)
    ei_la = jnp.asarray(edge_index_la).astype(jnp.int32)
    ei_lp = jnp.asarray(edge_index_la_pa).astype(jnp.int32)
    ei_pl = jnp.asarray(edge_index_pa_la).astype(jnp.int32)
    batch_pa = jnp.asarray(batch_pa).astype(jnp.int32)
    batch_la = jnp.asarray(batch_la).astype(jnp.int32)
    n_pa = x_pa.shape[0]; n_la = x_la.shape[0]

    p1 = params['l1']
    h_pa = _gate_conv(x_pa, x_pa, ei_pa, edge_attr_pa, p1['pa'], n_pa) + \
        _gate_conv(x_la, x_pa, ei_lp, edge_attr_la_pa, p1['la_pa'], n_pa)
    h_la = _gate_conv(x_la, x_la, ei_la, edge_attr_la, p1['la'], n_la) + \
        _gate_conv(x_pa, x_la, ei_pl, edge_attr_pa_la, p1['pa_la'], n_la)
    h_pa = jax.nn.relu(h_pa); h_la = jax.nn.relu(h_la)
    for lname in ('l2', 'l3'):
        pll = params[lname]
        new_pa = _gat_conv(h_pa, h_pa, ei_pa, pll['pa'], n_pa) + \
            _gat_conv(h_la, h_pa, ei_lp, pll['la_pa'], n_pa)
        new_la = _gat_conv(h_la, h_la, ei_la, pll['la'], n_la) + \
            _gat_conv(h_pa, h_la, ei_pl, pll['pa_la'], n_la)
        h_pa = jax.nn.relu(new_pa); h_la = jax.nn.relu(new_la)
    oh_pa = (batch_pa[:, None] == jnp.arange(NG, dtype=jnp.int32)).astype(jnp.float32)
    oh_la = (batch_la[:, None] == jnp.arange(NG, dtype=jnp.int32)).astype(jnp.float32)
    mol_pa = oh_pa.T @ h_pa
    mol_la = oh_la.T @ h_la
    q_pa = h_pa @ params['mol_pa']['Wsrc']; qa_pa = q_pa @ params['mol_pa']['a_src']
    q_la = h_la @ params['mol_la']['Wsrc']; qa_la = q_la @ params['mol_la']['a_src']
    for _ in range(3):
        mol_pa = _mol_conv(q_pa, qa_pa, oh_pa, mol_pa, params['mol_pa'], NG)
        mol_la = _mol_conv(q_la, qa_la, oh_la, mol_la, params['mol_la'], NG)
    return _final_mlp(mol_pa, mol_la, params)


# speedup vs baseline: 3.0950x; 1.0914x over previous
import functools

import jax
import jax.numpy as jnp
from jax import lax
from jax.experimental import pallas as pl
from jax.experimental.pallas import tpu as pltpu
from jax.experimental.pallas import tpu_sc as plsc

H = 64
NG = 128
NW = 32
CH = 128
BR = 512


@functools.lru_cache(maxsize=None)
def _make_sc_gather(E, D, N):
    SUB = 4
    BB = SUB * CH
    per_tile = E // NW
    nblocks = per_tile // BB
    mesh = plsc.VectorSubcoreMesh(core_axis_name="c", subcore_axis_name="s")

    @functools.partial(
        pl.kernel, mesh=mesh,
        out_type=jax.ShapeDtypeStruct((E, D), jnp.float32),
        compiler_params=pltpu.CompilerParams(use_tc_tiling_on_sc=False),
        scratch_types=[
            pltpu.VMEM((SUB, CH), jnp.int32),
            pltpu.VMEM((BB, D), jnp.float32),
            pltpu.SemaphoreType.DMA,
        ],
    )
    def gather_k(table_hbm, idx_hbm, out_hbm, idx_v, rows_v, sem):
        wid = lax.axis_index("s") * 2 + lax.axis_index("c")
        base0 = wid * per_tile

        def body(b, carry):
            base = base0 + b * BB
            pltpu.sync_copy(idx_hbm.at[pl.ds(base // CH, SUB)], idx_v)
            cps = [pltpu.async_copy(table_hbm.at[idx_v.at[j]],
                                    rows_v.at[pl.ds(j * CH, CH)], sem)
                   for j in range(SUB)]
            for cp in cps:
                cp.wait()
            pltpu.sync_copy(rows_v, out_hbm.at[pl.ds(base, BB)])
            return carry

        lax.fori_loop(0, nblocks, body, 0)

    return gather_k


def _sc_gather(table, idx_pad):
    E = idx_pad.shape[0]
    N, D = table.shape
    return _make_sc_gather(E, D, N)(table, idx_pad.reshape(E // CH, CH))


def _acc_rows(n):
    return -(-(n + 1) // (16 * CH)) * (16 * CH)


@functools.lru_cache(maxsize=None)
def _make_sc_scatter_add(E, N, D):
    NP = _acc_rows(N)
    SUB = 4
    BB = SUB * CH
    per_tile = E // 16
    nblocks = per_tile // BB
    zcopies = NP // 16 // CH
    mesh = plsc.VectorSubcoreMesh(core_axis_name="c", subcore_axis_name="s")

    @functools.partial(
        pl.kernel, mesh=mesh,
        out_type=jax.ShapeDtypeStruct((2 * NP, D), jnp.float32),
        compiler_params=pltpu.CompilerParams(use_tc_tiling_on_sc=False),
        scratch_types=(
            pltpu.VMEM((SUB, CH), jnp.int32),
            pltpu.VMEM((BB, D), jnp.float32),
            pltpu.VMEM((CH, D), jnp.float32),
            pltpu.VMEM_SHARED((NP, D), jnp.float32),
            pltpu.SemaphoreType.DMA,
        ),
    )
    def scatter_k(vals_hbm, idx_hbm, out_hbm, idx_v, val_v, zero_v, acc_sh, sem):
        core = lax.axis_index("c")
        sub = lax.axis_index("s")

        def zrow(r, carry):
            for j in range(D // 16):
                zero_v[r, pl.ds(j * 16, 16)] = jnp.zeros((16,), jnp.float32)
            return carry
        lax.fori_loop(0, CH, zrow, 0)

        zbase = sub * (NP // 16)
        def zacc(i, carry):
            pltpu.sync_copy(zero_v, acc_sh.at[pl.ds(zbase + i * CH, CH)])
            return carry
        lax.fori_loop(0, zcopies, zacc, 0)
        plsc.subcore_barrier()

        def body(b, carry):
            base = sub * per_tile + b * BB
            pltpu.sync_copy(idx_hbm.at[pl.ds(base // CH, SUB)], idx_v)
            pltpu.sync_copy(vals_hbm.at[pl.ds(core * E + base, BB)], val_v)
            cps = [pltpu.async_copy(val_v.at[pl.ds(j * CH, CH)],
                                    acc_sh.at[idx_v.at[j]], sem, add=True)
                   for j in range(SUB)]
            for cp in cps:
                cp.wait()
            return carry
        lax.fori_loop(0, nblocks, body, 0)
        plsc.subcore_barrier()

        def cpout(i, carry):
            sl = pl.ds(zbase + i * BB, BB)
            pltpu.sync_copy(acc_sh.at[sl], val_v)
            pltpu.sync_copy(val_v, out_hbm.at[pl.ds(core * NP + zbase + i * BB, BB)])
            return carry
        lax.fori_loop(0, zcopies // SUB, cpout, 0)
        for r in range(zcopies % SUB):
            off = (zcopies // SUB) * BB + r * CH
            pltpu.sync_copy(acc_sh.at[pl.ds(zbase + off, CH)], val_v.at[pl.ds(0, CH)])
            pltpu.sync_copy(val_v.at[pl.ds(0, CH)],
                            out_hbm.at[pl.ds(core * NP + zbase + off, CH)])

    return scatter_k


def _sc_segment_sum32(vals2, dst_pad, n):
    Ep = vals2.shape[0] // 2
    NP = _acc_rows(n)
    acc = _make_sc_scatter_add(Ep, n, 32)(vals2, dst_pad.reshape(Ep // CH, CH))
    return jnp.concatenate([acc[:n], acc[NP:NP + n]], axis=1)


def _sc_segment_sum16(w16_2, dst_pad, n):
    Ep = w16_2.shape[0] // 2
    dacc = _make_sc_scatter_add(Ep, n, 16)(w16_2, dst_pad.reshape(Ep // CH, CH))
    return dacc[:n, :1]


def _pad_edges(ei, n_dst):
    E = ei.shape[1]
    Ep = -(-E // 16384) * 16384
    src = jnp.concatenate([ei[0], jnp.zeros((Ep - E,), jnp.int32)])
    dst = jnp.concatenate([ei[1], jnp.full((Ep - E,), n_dst, jnp.int32)])
    return src, dst, E, Ep


def _lin_body(x_ref, w_ref, o_ref):
    o_ref[...] = jnp.dot(x_ref[...], w_ref[...],
                         preferred_element_type=jnp.float32)


def _tc_lin(x, w):
    R, Kin = x.shape
    Kout = w.shape[1]
    return pl.pallas_call(
        _lin_body,
        grid=(R // BR,),
        in_specs=[pl.BlockSpec((BR, Kin), lambda i: (i, 0)),
                  pl.BlockSpec((Kin, Kout), lambda i: (0, 0))],
        out_specs=pl.BlockSpec((BR, Kout), lambda i: (i, 0)),
        out_shape=jax.ShapeDtypeStruct((R, Kout), jnp.float32),
    )(x, w)


def _gru_block(h, m, W, U, bi, bh):
    gi = jnp.dot(m, W, preferred_element_type=jnp.float32) + bi
    gh = jnp.dot(h, U, preferred_element_type=jnp.float32) + bh
    i_r, i_z, i_n = gi[:, :H], gi[:, H:2 * H], gi[:, 2 * H:]
    h_r, h_z, h_n = gh[:, :H], gh[:, H:2 * H], gh[:, 2 * H:]
    r = jax.nn.sigmoid(i_r + h_r)
    z = jax.nn.sigmoid(i_z + h_z)
    nn = jnp.tanh(i_n + r * h_n)
    return (1.0 - z) * nn + z * h


def _gru2_body(h1_ref, n1_ref, d1_ref, h2_ref, n2_ref, d2_ref,
               w1_ref, u1_ref, bi1_ref, bh1_ref,
               w2_ref, u2_ref, bi2_ref, bh2_ref, o_ref):
    m1 = n1_ref[...] / (d1_ref[...] + 1e-16)
    m2 = n2_ref[...] / (d2_ref[...] + 1e-16)
    g1 = _gru_block(h1_ref[...], m1, w1_ref[...], u1_ref[...],
                    bi1_ref[...], bh1_ref[...])
    g2 = _gru_block(h2_ref[...], m2, w2_ref[...], u2_ref[...],
                    bi2_ref[...], bh2_ref[...])
    o_ref[...] = jnp.maximum(g1 + g2, 0.0)


def _tc_gru2(h1, num1, den1, p1, h2, num2, den2, p2):
    n = h1.shape[0]
    R = -(-n // BR) * BR
    h1 = _pad_rows(h1, R); num1 = _pad_rows(num1, R); den1 = _pad_rows(den1, R)
    h2 = _pad_rows(h2, R); num2 = _pad_rows(num2, R); den2 = _pad_rows(den2, R)
    bspec = pl.BlockSpec((BR, H), lambda i: (i, 0))
    dspec = pl.BlockSpec((BR, 1), lambda i: (i, 0))
    wspec = pl.BlockSpec((H, 3 * H), lambda i: (0, 0))
    bsp = pl.BlockSpec((1, 3 * H), lambda i: (0, 0))
    out = pl.pallas_call(
        _gru2_body,
        grid=(R // BR,),
        in_specs=[bspec, bspec, dspec, bspec, bspec, dspec,
                  wspec, wspec, bsp, bsp, wspec, wspec, bsp, bsp],
        out_specs=bspec,
        out_shape=jax.ShapeDtypeStruct((R, H), jnp.float32),
    )(h1, num1, den1, h2, num2, den2,
      p1['W'], p1['U'], p1['bi'].reshape(1, -1), p1['bh'].reshape(1, -1),
      p2['W'], p2['U'], p2['bi'].reshape(1, -1), p2['bh'].reshape(1, -1))
    return out[:n]


def _edge_gate_body(sg_ref, ea_ref, we_ref, gw_ref, gb_ref, mw_ref, mb_ref,
                    o_ref):
    xj = sg_ref[...] + jnp.dot(ea_ref[...], we_ref[...],
                               preferred_element_type=jnp.float32)
    gate = jax.nn.sigmoid(
        jnp.dot(xj, gw_ref[...], preferred_element_type=jnp.float32)
        + gb_ref[...])
    msg = gate * jnp.tanh(
        jnp.dot(xj, mw_ref[...], preferred_element_type=jnp.float32)
        + mb_ref[...])
    o_ref[0, :, :] = msg[:, :32]
    o_ref[1, :, :] = msg[:, 32:]


def _tc_edge_gate(sg, ea16, we16, gw, gb, mw, mb):
    Ep = sg.shape[0]
    return pl.pallas_call(
        _edge_gate_body,
        grid=(Ep // BR,),
        in_specs=[pl.BlockSpec((BR, H), lambda i: (i, 0)),
                  pl.BlockSpec((BR, 16), lambda i: (i, 0)),
                  pl.BlockSpec((16, H), lambda i: (0, 0)),
                  pl.BlockSpec((H, H), lambda i: (0, 0)),
                  pl.BlockSpec((1, H), lambda i: (0, 0)),
                  pl.BlockSpec((H, H), lambda i: (0, 0)),
                  pl.BlockSpec((1, H), lambda i: (0, 0))],
        out_specs=pl.BlockSpec((2, BR, 32), lambda i: (0, i, 0)),
        out_shape=jax.ShapeDtypeStruct((2, Ep, 32), jnp.float32),
    )(sg, ea16, we16, gw, gb, mw, mb)


def _edge_gat_body(qg_ref, kg_ref, v_ref, w_ref):
    qg = qg_ref[...]
    logit = qg[:, 64:65] + kg_ref[...][:, :1]
    w = jnp.exp(jnp.where(logit >= 0.0, logit, 0.2 * logit))
    v_ref[0, :, :] = w * qg[:, :32]
    v_ref[1, :, :] = w * qg[:, 32:64]
    w16 = jnp.concatenate([w, jnp.zeros((BR, 15), jnp.float32)], axis=1)
    w_ref[0, :, :] = w16
    w_ref[1, :, :] = w16


def _tc_edge_gat(qg, kg):
    Ep = qg.shape[0]
    return pl.pallas_call(
        _edge_gat_body,
        grid=(Ep // BR,),
        in_specs=[pl.BlockSpec((BR, 80), lambda i: (i, 0)),
                  pl.BlockSpec((BR, 16), lambda i: (i, 0))],
        out_specs=[pl.BlockSpec((2, BR, 32), lambda i: (0, i, 0)),
                   pl.BlockSpec((2, BR, 16), lambda i: (0, i, 0))],
        out_shape=[jax.ShapeDtypeStruct((2, Ep, 32), jnp.float32),
                   jax.ShapeDtypeStruct((2, Ep, 16), jnp.float32)],
    )(qg, kg)


def _seg_body(h_ref, b_ref, o_ref):
    i = pl.program_id(0)
    oh = (b_ref[0, 0, :][:, None]
          == jax.lax.broadcasted_iota(jnp.int32, (BR, NG), 1)
          ).astype(jnp.float32)
    part = jnp.dot(oh.T, h_ref[...], preferred_element_type=jnp.float32)

    @pl.when(i == 0)
    def _():
        o_ref[...] = jnp.zeros_like(o_ref)
    o_ref[...] += part


def _tc_seg_sum(h_pad, batch3):
    R = h_pad.shape[0]
    return pl.pallas_call(
        _seg_body,
        grid=(R // BR,),
        in_specs=[pl.BlockSpec((BR, H), lambda i: (i, 0)),
                  pl.BlockSpec((1, 1, BR), lambda i: (i, 0, 0))],
        out_specs=pl.BlockSpec((NG, H), lambda i: (0, 0)),
        out_shape=jax.ShapeDtypeStruct((NG, H), jnp.float32),
    )(h_pad, batch3)


def _mol_body(qe_ref, b_ref, mol_ref, wd_ref, ad_ref,
              w_ref, u_ref, bi_ref, bh_ref, o_ref, acc_ref):
    i = pl.program_id(0)
    nb = pl.num_programs(0)

    @pl.when(i == 0)
    def _():
        acc_ref[...] = jnp.zeros_like(acc_ref)

    kb = jnp.dot(jnp.dot(mol_ref[...], wd_ref[...],
                         preferred_element_type=jnp.float32),
                 ad_ref[...], preferred_element_type=jnp.float32)
    oh = (b_ref[0, 0, :][:, None]
          == jax.lax.broadcasted_iota(jnp.int32, (BR, NG), 1)
          ).astype(jnp.float32)
    qe = qe_ref[...]
    logit = qe[:, 64:65] + jnp.dot(oh, kb, preferred_element_type=jnp.float32)
    w = jnp.exp(jnp.where(logit >= 0.0, logit, 0.2 * logit))
    vals = jnp.concatenate(
        [w * qe[:, :64], w, jnp.zeros((BR, 15), jnp.float32)], axis=1)
    acc_ref[...] += jnp.dot(oh.T, vals, preferred_element_type=jnp.float32)

    @pl.when(i == nb - 1)
    def _():
        acc = acc_ref[...]
        agg = acc[:, :H] / (acc[:, H:H + 1] + 1e-16)
        o_ref[...] = _gru_block(mol_ref[...], agg, w_ref[...], u_ref[...],
                                bi_ref[...], bh_ref[...])


def _tc_mol_iter(qe_mol, batch3, mol, p):
    R = qe_mol.shape[0]

    def cspec(shape):
        return pl.BlockSpec(shape, lambda i: tuple(0 for _ in shape))

    return pl.pallas_call(
        _mol_body,
        grid=(R // BR,),
        in_specs=[pl.BlockSpec((BR, 80), lambda i: (i, 0)),
                  pl.BlockSpec((1, 1, BR), lambda i: (i, 0, 0)),
                  cspec((NG, H)), cspec((H, H)), cspec((H, 1)),
                  cspec((H, 3 * H)), cspec((H, 3 * H)),
                  cspec((1, 3 * H)), cspec((1, 3 * H))],
        out_specs=cspec((NG, H)),
        out_shape=jax.ShapeDtypeStruct((NG, H), jnp.float32),
        scratch_shapes=[pltpu.VMEM((NG, 80), jnp.float32)],
    )(qe_mol, batch3, mol, p['Wdst'], p['a_dst'].reshape(H, 1),
      p['gru']['W'], p['gru']['U'],
      p['gru']['bi'].reshape(1, -1), p['gru']['bh'].reshape(1, -1))


def _final_mlp_body(mol_pa_ref, mol_la_ref, wpa_ref, bpa_ref, wla_ref, bla_ref,
                    w0_ref, b0_ref, w1_ref, b1_ref, out_ref):
    y_pa = mol_pa_ref[...] @ wpa_ref[...] + bpa_ref[...]
    y_la = mol_la_ref[...] @ wla_ref[...] + bla_ref[...]
    z = jnp.concatenate([y_pa, y_la], axis=-1)
    z = jax.nn.relu(z @ w0_ref[...] + b0_ref[...])
    out_ref[...] = z @ w1_ref[...] + b1_ref[...]


def _final_mlp(mol_pa, mol_la, params):
    return pl.pallas_call(
        _final_mlp_body,
        out_shape=jax.ShapeDtypeStruct((NG, 1), jnp.float32),
    )(mol_pa, mol_la,
      params['lin_pa']['w'], params['lin_pa']['b'].reshape(1, H),
      params['lin_la']['w'], params['lin_la']['b'].reshape(1, H),
      params['mlp0']['w'], params['mlp0']['b'].reshape(1, H),
      params['mlp1']['w'], params['mlp1']['b'].reshape(1, 1))


def _pad_rows(x, rows):
    return jnp.concatenate(
        [x, jnp.zeros((rows - x.shape[0], x.shape[1]), jnp.float32)], axis=0)


def _gate_conv(x64_src, x64_dst, ei, ea, p, n_dst):
    src, dst, E, Ep = _pad_edges(ei, n_dst)
    Wlin = jnp.pad(p['lin'], ((0, 64 - p['lin'].shape[0]), (0, 0)))
    Wsrc = jnp.pad(p['src'], ((0, 64 - p['src'].shape[0]), (0, 0)))
    h = _tc_lin(x64_dst, Wlin)
    s = _tc_lin(x64_src, Wsrc)
    sg = _sc_gather(s, src)
    ea16 = jnp.pad(ea, ((0, Ep - E), (0, 16 - ea.shape[1])))
    we16 = jnp.pad(p['edge'], ((0, 16 - p['edge'].shape[0]), (0, 0)))
    vals = _tc_edge_gate(sg, ea16, we16, p['gate_w'],
                         p['gate_b'].reshape(1, H), p['msg_w'],
                         p['msg_b'].reshape(1, H))
    num = _sc_segment_sum32(vals.reshape(2 * Ep, 32), dst, n_dst)
    return h, num


def _gat_edge(h_src_pad, h_dst_pad, ei, p, n_dst):
    src, dst, E, Ep = _pad_edges(ei, n_dst)
    wq = jnp.concatenate(
        [p['Wsrc'], (p['Wsrc'] @ p['a_src'])[:, None],
         jnp.zeros((H, 15), jnp.float32)], axis=1)
    wk = jnp.concatenate(
        [(p['Wdst'] @ p['a_dst'])[:, None], jnp.zeros((H, 15), jnp.float32)],
        axis=1)
    qext = _tc_lin(h_src_pad, wq)
    kext = _tc_lin(h_dst_pad, wk)
    qg = _sc_gather(qext, src)
    kg = _sc_gather(kext, dst)
    vals, w16 = _tc_edge_gat(qg, kg)
    num = _sc_segment_sum32(vals.reshape(2 * Ep, 32), dst, n_dst)
    den = _sc_segment_sum16(w16.reshape(2 * Ep, 16), dst, n_dst)
    return num, den


def kernel(x_pa, x_la, edge_index_pa, edge_index_la, edge_index_la_pa, edge_index_pa_la,
           edge_attr_pa, edge_attr_la, edge_attr_la_pa, edge_attr_pa_la,
           batch_pa, batch_la, params):
    ei_pa = jnp.asarray(edge_index_pa).astype(jnp.int32)
    ei_la = jnp.asarray(edge_index_la).astype(jnp.int32)
    ei_lp = jnp.asarray(edge_index_la_pa).astype(jnp.int32)
    ei_pl = jnp.asarray(edge_index_pa_la).astype(jnp.int32)
    n_pa = x_pa.shape[0]; n_la = x_la.shape[0]
    NPAD = -(-max(n_pa, n_la) // BR) * BR
    x64_pa = _pad_rows(jnp.pad(x_pa, ((0, 0), (0, 64 - x_pa.shape[1]))), NPAD)
    x64_la = _pad_rows(jnp.pad(x_la, ((0, 0), (0, 64 - x_la.shape[1]))), NPAD)
    ones_pa = jnp.ones((n_pa, 1), jnp.float32)
    ones_la = jnp.ones((n_la, 1), jnp.float32)
    batch3_pa = jnp.concatenate(
        [jnp.asarray(batch_pa).astype(jnp.int32),
         jnp.full((NPAD - n_pa,), NG, jnp.int32)]).reshape(NPAD // BR, 1, BR)
    batch3_la = jnp.concatenate(
        [jnp.asarray(batch_la).astype(jnp.int32),
         jnp.full((NPAD - n_la,), NG, jnp.int32)]).reshape(NPAD // BR, 1, BR)

    p1 = params['l1']
    h1, num1 = _gate_conv(x64_pa, x64_pa, ei_pa, edge_attr_pa, p1['pa'], n_pa)
    h2, num2 = _gate_conv(x64_la, x64_pa, ei_lp, edge_attr_la_pa, p1['la_pa'], n_pa)
    h_pa = _tc_gru2(h1[:n_pa], num1, ones_pa, p1['pa']['gru'],
                    h2[:n_pa], num2, ones_pa, p1['la_pa']['gru'])
    h1, num1 = _gate_conv(x64_la, x64_la, ei_la, edge_attr_la, p1['la'], n_la)
    h2, num2 = _gate_conv(x64_pa, x64_la, ei_pl, edge_attr_pa_la, p1['pa_la'], n_la)
    h_la = _tc_gru2(h1[:n_la], num1, ones_la, p1['la']['gru'],
                    h2[:n_la], num2, ones_la, p1['pa_la']['gru'])

    for lname in ('l2', 'l3'):
        pll = params[lname]
        hp = _pad_rows(h_pa, NPAD)
        hl = _pad_rows(h_la, NPAD)
        na, da = _gat_edge(hp, hp, ei_pa, pll['pa'], n_pa)
        nb, db = _gat_edge(hl, hp, ei_lp, pll['la_pa'], n_pa)
        nc, dc = _gat_edge(hl, hl, ei_la, pll['la'], n_la)
        nd, dd = _gat_edge(hp, hl, ei_pl, pll['pa_la'], n_la)
        h_pa = _tc_gru2(h_pa, na, da, pll['pa']['gru'],
                        h_pa, nb, db, pll['la_pa']['gru'])
        h_la = _tc_gru2(h_la, nc, dc, pll['la']['gru'],
                        h_la, nd, dd, pll['pa_la']['gru'])

    hp = _pad_rows(h_pa, NPAD)
    hl = _pad_rows(h_la, NPAD)
    mol_pa = _tc_seg_sum(hp, batch3_pa)
    mol_la = _tc_seg_sum(hl, batch3_la)
    wq_pa = jnp.concatenate(
        [params['mol_pa']['Wsrc'],
         (params['mol_pa']['Wsrc'] @ params['mol_pa']['a_src'])[:, None],
         jnp.zeros((H, 15), jnp.float32)], axis=1)
    wq_la = jnp.concatenate(
        [params['mol_la']['Wsrc'],
         (params['mol_la']['Wsrc'] @ params['mol_la']['a_src'])[:, None],
         jnp.zeros((H, 15), jnp.float32)], axis=1)
    qe_pa = _tc_lin(hp, wq_pa)
    qe_la = _tc_lin(hl, wq_la)
    for _ in range(3):
        mol_pa = _tc_mol_iter(qe_pa, batch3_pa, mol_pa, params['mol_pa'])
        mol_la = _tc_mol_iter(qe_la, batch3_la, mol_la, params['mol_la'])
    return _final_mlp(mol_pa, mol_la, params)
